# baseline (speedup 1.0000x reference)
import functools

import jax
import jax.numpy as jnp
from jax import lax
from jax.experimental import pallas as pl
from jax.experimental.pallas import tpu as pltpu
from jax.experimental.pallas import tpu_sc as plsc

NUM_NODES = 100000
NUM_SEGMENTS = 2048
FEATURE_DIM = 128

NUM_CHUNKS = 4
NUM_FGROUPS = 8
ROWS_PER_CHUNK = NUM_NODES // NUM_CHUNKS
TILE_ROWS = 1000
TILES_PER_CHUNK = ROWS_PER_CHUNK // TILE_ROWS
FULL_GROUPS = TILE_ROWS // 16
TAIL_ROWS = TILE_ROWS - FULL_GROUPS * 16
SEG_SLICE = NUM_SEGMENTS // NUM_CHUNKS


def _sc_body(x_hbm, batch_hbm, out_hbm,
             xb0, xb1, ib0, ib1, acc, cnt, cbuf, rbuf, obuf, inv,
             shacc, shcnt,
             sx0, sx1, si0, si1):
    nc = 2
    core = lax.axis_index("c")
    sub = lax.axis_index("s")
    wid = sub * nc + core
    chunk = wid // NUM_FGROUPS
    q = wid % NUM_FGROUPS
    f0 = q * 16
    row_base = chunk * ROWS_PER_CHUNK

    xbufs = (xb0, xb1)
    ibufs = (ib0, ib1)
    xsems = (sx0, sx1)
    isems = (si0, si1)

    def start(k):
        slot = k % 2
        row0 = row_base + k * TILE_ROWS
        cx = pltpu.async_copy(
            x_hbm.at[pl.ds(row0, TILE_ROWS), pl.ds(f0, 16)],
            xbufs[slot], xsems[slot])
        ci = pltpu.async_copy(
            batch_hbm.at[pl.ds(row0, TILE_ROWS)],
            ibufs[slot], isems[slot])
        return cx, ci

    handles = {0: start(0)}

    zf = jnp.zeros((16,), jnp.float32)

    def zacc(i, _):
        acc[i] = zf
        return _

    lax.fori_loop(0, NUM_SEGMENTS, zacc, None)

    def zcnt(i, _):
        cnt[pl.ds(i * 16, 16)] = zf
        return _

    lax.fori_loop(0, NUM_SEGMENTS // 16, zcnt, None)

    lanes = lax.iota(jnp.int32, 16)

    def flush(cur, accv, rn):
        plsc.addupdate(acc.at[cur], accv)
        base = (cur >> 4) << 4
        oh = jnp.where(lanes == (cur & 15), rn, jnp.float32(0.0))
        plsc.addupdate(cnt.at[pl.ds(base, 16)], oh)

    carry = (jnp.int32(0), zf, jnp.float32(0.0))

    for k in range(TILES_PER_CHUNK):
        slot = k % 2
        if k + 1 < TILES_PER_CHUNK:
            handles[k + 1] = start(k + 1)
        cx, ci = handles.pop(k)
        cx.wait()
        ci.wait()
        xb = xbufs[slot]
        ib = ibufs[slot]

        def row_step(c, sid, xrow):
            cur, accv, rn = c
            change = sid != cur

            @pl.when(change)
            def _():
                flush(cur, accv, rn)

            accv = jnp.where(change, xrow, accv + xrow)
            rn = jnp.where(change, jnp.float32(1.0), rn + jnp.float32(1.0))
            return (sid, accv, rn)

        def group(i, c):
            ids16 = ib[pl.ds(i * 16, 16)]
            base = i * 16
            cur, accv, rn = c
            uniform = (ids16[0] == cur) & (ids16[15] == cur)

            def fast(c2):
                cur2, accv2, rn2 = c2
                s = accv2
                for u in range(16):
                    s = s + xb[base + u]
                return (cur2, s, rn2 + jnp.float32(16.0))

            def slow(c2):
                for u in range(16):
                    c2 = row_step(c2, ids16[u], xb[base + u])
                return c2

            return lax.cond(uniform, fast, slow, c)

        carry = lax.fori_loop(0, FULL_GROUPS, group, carry)

        tail_ids = ib[pl.ds(TILE_ROWS - 16, 16)]
        for u in range(16 - TAIL_ROWS, 16):
            r = TILE_ROWS - 16 + u
            carry = row_step(carry, tail_ids[u], xb[r])

    cur, accv, rn = carry
    flush(cur, accv, rn)

    pltpu.sync_copy(acc, shacc.at[sub])

    @pl.when(q == core)
    def _():
        pltpu.sync_copy(cnt, shcnt.at[chunk])

    plsc.subcore_barrier()

    pltpu.sync_copy(shcnt, cbuf)
    sg0 = chunk * SEG_SLICE

    one = jnp.float32(1.0)

    def invf(i, _):
        s = pl.ds(i * 16, 16)
        c = cbuf[0, s] + cbuf[1, s] + cbuf[2, s] + cbuf[3, s]
        inv[s] = one / c
        return _

    lax.fori_loop(0, NUM_SEGMENTS // 16, invf, None)

    for j in range(NUM_CHUNKS):
        pltpu.sync_copy(
            shacc.at[j * NUM_CHUNKS + q // 2, pl.ds(sg0, SEG_SLICE)],
            rbuf)

        if j == 0:
            def acc0(i, _):
                obuf[i] = rbuf[i]
                return _
            lax.fori_loop(0, SEG_SLICE, acc0, None)
        elif j < NUM_CHUNKS - 1:
            def accj(i, _):
                obuf[i] = obuf[i] + rbuf[i]
                return _
            lax.fori_loop(0, SEG_SLICE, accj, None)
        else:
            def accl(b, _):
                inv16 = inv[pl.ds(sg0 + b * 16, 16)]
                for u in range(16):
                    i = b * 16 + u
                    obuf[i] = (obuf[i] + rbuf[i]) * inv16[u]
                return _
            lax.fori_loop(0, SEG_SLICE // 16, accl, None)

    pltpu.async_copy(
        obuf, out_hbm.at[pl.ds(sg0, SEG_SLICE), pl.ds(f0, 16)], sx0).wait()


def _segment_mean(x, batch):
    mesh = plsc.VectorSubcoreMesh(core_axis_name="c", subcore_axis_name="s")
    k = functools.partial(
        pl.kernel,
        out_type=jax.ShapeDtypeStruct((NUM_SEGMENTS, FEATURE_DIM),
                                      jnp.float32),
        mesh=mesh,
        compiler_params=pltpu.CompilerParams(use_tc_tiling_on_sc=False,
                                             needs_layout_passes=False),
        scratch_types=[
            pltpu.VMEM((TILE_ROWS, 16), jnp.float32),
            pltpu.VMEM((TILE_ROWS, 16), jnp.float32),
            pltpu.VMEM((TILE_ROWS,), jnp.int32),
            pltpu.VMEM((TILE_ROWS,), jnp.int32),
            pltpu.VMEM((NUM_SEGMENTS, 16), jnp.float32),
            pltpu.VMEM((NUM_SEGMENTS,), jnp.float32),
            pltpu.VMEM((NUM_CHUNKS, NUM_SEGMENTS), jnp.float32),
            pltpu.VMEM((SEG_SLICE, 16), jnp.float32),
            pltpu.VMEM((SEG_SLICE, 16), jnp.float32),
            pltpu.VMEM((NUM_SEGMENTS,), jnp.float32),
            pltpu.VMEM_SHARED((16, NUM_SEGMENTS, 16), jnp.float32),
            pltpu.VMEM_SHARED((NUM_CHUNKS, NUM_SEGMENTS), jnp.float32),
            pltpu.SemaphoreType.DMA,
            pltpu.SemaphoreType.DMA,
            pltpu.SemaphoreType.DMA,
            pltpu.SemaphoreType.DMA,
        ],
    )(_sc_body)
    return k(x, batch)


def kernel(x, batch, grid_size):
    del grid_size
    return _segment_mean(x, batch.astype(jnp.int32))


# speedup vs baseline: 1.7652x; 1.7652x over previous
import functools

import jax
import jax.numpy as jnp
from jax import lax
from jax.experimental import pallas as pl
from jax.experimental.pallas import tpu as pltpu
from jax.experimental.pallas import tpu_sc as plsc

NUM_NODES = 100000
NUM_SEGMENTS = 2048
FEATURE_DIM = 128

NUM_CHUNKS = 4
NUM_FGROUPS = 8
ROWS_PER_CHUNK = NUM_NODES // NUM_CHUNKS
TILE_ROWS = 1000
TILES_PER_CHUNK = ROWS_PER_CHUNK // TILE_ROWS
FULL_GROUPS = TILE_ROWS // 16
TAIL_ROWS = TILE_ROWS - FULL_GROUPS * 16
NUM_GROUPS = FULL_GROUPS + 1
ZBUF_ROWS = 256
SEG_SLICE = NUM_SEGMENTS // NUM_CHUNKS


def _sc_body(x_hbm, batch_hbm, out_hbm,
             xb0, xb1, ib0, ib1, zbuf, cnt, cbuf, rbuf, obuf, inv,
             shacc, shcnt,
             sx0, sx1, si0, si1, ss0, ss1):
    nc = 2
    core = lax.axis_index("c")
    sub = lax.axis_index("s")
    wid = sub * nc + core
    chunk = wid // NUM_FGROUPS
    q = wid % NUM_FGROUPS
    f0 = q * 16
    row_base = chunk * ROWS_PER_CHUNK

    xbufs = (xb0, xb1)
    ibufs = (ib0, ib1)
    xsems = (sx0, sx1)
    isems = (si0, si1)
    ssems = (ss0, ss1)

    def start(k):
        slot = k % 2
        row0 = row_base + k * TILE_ROWS
        cx = pltpu.async_copy(
            x_hbm.at[pl.ds(row0, TILE_ROWS), pl.ds(f0, 16)],
            xbufs[slot], xsems[slot])
        ci = pltpu.async_copy(
            batch_hbm.at[pl.ds(row0, TILE_ROWS)],
            ibufs[slot], isems[slot])
        return cx, ci

    handles = {0: start(0)}

    plane = shacc.at[sub]

    zf = jnp.zeros((16,), jnp.float32)

    def zbz(i, _):
        zbuf[i] = zf
        return _

    lax.fori_loop(0, ZBUF_ROWS, zbz, None)
    for j in range(NUM_SEGMENTS // ZBUF_ROWS):
        pltpu.sync_copy(zbuf, plane.at[pl.ds(j * ZBUF_ROWS, ZBUF_ROWS)])

    def zcnt(i, _):
        cnt[pl.ds(i * 16, 16)] = zf
        return _

    lax.fori_loop(0, NUM_SEGMENTS // 16, zcnt, None)

    lanes = lax.iota(jnp.int32, 16)
    ones16 = jnp.ones((16,), jnp.float32)

    scat = {}
    for k in range(TILES_PER_CHUNK):
        slot = k % 2
        if k - 1 >= 0:
            scat.pop(k - 1).wait()
        if k + 1 < TILES_PER_CHUNK:
            handles[k + 1] = start(k + 1)
        cx, ci = handles.pop(k)
        cx.wait()
        ci.wait()
        xb = xbufs[slot]
        ib = ibufs[slot]

        scat[k] = pltpu.async_copy(xb, plane.at[ib], ssems[slot], add=True)

        def cgroup(g, _):
            is_tail = g == FULL_GROUPS
            off = jnp.where(is_tail, TILE_ROWS - 16, g * 16)
            lo = jnp.where(is_tail, 16 - TAIL_ROWS, 0)
            ids16 = ib[pl.ds(off, 16)]
            mask = lanes >= lo
            idx = jnp.where(mask, ids16, 0)
            plsc.addupdate_scatter(cnt, [idx], ones16, mask=mask)
            return _

        lax.fori_loop(0, NUM_GROUPS, cgroup, None)
    scat.pop(TILES_PER_CHUNK - 1).wait()

    @pl.when(q == core)
    def _():
        pltpu.sync_copy(cnt, shcnt.at[chunk])

    plsc.subcore_barrier()

    pltpu.sync_copy(shcnt, cbuf)
    sg0 = chunk * SEG_SLICE
    for j in range(NUM_CHUNKS):
        pltpu.sync_copy(
            shacc.at[j * NUM_CHUNKS + q // 2, pl.ds(sg0, SEG_SLICE)],
            rbuf.at[j])

    one = jnp.float32(1.0)

    def invf(i, _):
        s = pl.ds(i * 16, 16)
        c = cbuf[0, s] + cbuf[1, s] + cbuf[2, s] + cbuf[3, s]
        inv[s] = one / c
        return _

    lax.fori_loop(0, NUM_SEGMENTS // 16, invf, None)

    def blockf(b, _):
        inv16 = inv[pl.ds(sg0 + b * 16, 16)]
        for u in range(16):
            i = b * 16 + u
            v = rbuf[0, i] + rbuf[1, i] + rbuf[2, i] + rbuf[3, i]
            obuf[i] = v * inv16[u]
        return _

    lax.fori_loop(0, SEG_SLICE // 16, blockf, None)

    pltpu.async_copy(
        obuf, out_hbm.at[pl.ds(sg0, SEG_SLICE), pl.ds(f0, 16)], sx0).wait()


def _segment_mean(x, batch):
    mesh = plsc.VectorSubcoreMesh(core_axis_name="c", subcore_axis_name="s")
    k = functools.partial(
        pl.kernel,
        out_type=jax.ShapeDtypeStruct((NUM_SEGMENTS, FEATURE_DIM),
                                      jnp.float32),
        mesh=mesh,
        compiler_params=pltpu.CompilerParams(use_tc_tiling_on_sc=False,
                                             needs_layout_passes=False),
        scratch_types=[
            pltpu.VMEM((TILE_ROWS, 16), jnp.float32),
            pltpu.VMEM((TILE_ROWS, 16), jnp.float32),
            pltpu.VMEM((TILE_ROWS,), jnp.int32),
            pltpu.VMEM((TILE_ROWS,), jnp.int32),
            pltpu.VMEM((ZBUF_ROWS, 16), jnp.float32),
            pltpu.VMEM((NUM_SEGMENTS,), jnp.float32),
            pltpu.VMEM((NUM_CHUNKS, NUM_SEGMENTS), jnp.float32),
            pltpu.VMEM((NUM_CHUNKS, SEG_SLICE, 16), jnp.float32),
            pltpu.VMEM((SEG_SLICE, 16), jnp.float32),
            pltpu.VMEM((NUM_SEGMENTS,), jnp.float32),
            pltpu.VMEM_SHARED((16, NUM_SEGMENTS, 16), jnp.float32),
            pltpu.VMEM_SHARED((NUM_CHUNKS, NUM_SEGMENTS), jnp.float32),
            pltpu.SemaphoreType.DMA,
            pltpu.SemaphoreType.DMA,
            pltpu.SemaphoreType.DMA,
            pltpu.SemaphoreType.DMA,
            pltpu.SemaphoreType.DMA,
            pltpu.SemaphoreType.DMA,
        ],
    )(_sc_body)
    return k(x, batch)


def kernel(x, batch, grid_size):
    del grid_size
    return _segment_mean(x, batch.astype(jnp.int32))
